# baseline (speedup 1.0000x reference)
import jax
import jax.numpy as jnp
from jax.experimental import pallas as pl

B = 16384
D = 128
N_NT = 16
R = 16
BLK = 8192


def _body(x_ref, nt_ref, w_ref, idx_ref, o_ref):
    idx = idx_ref[...]
    w = w_ref[...]
    nt_ids = jax.lax.broadcasted_iota(jnp.int32, (R, N_NT), 1)
    sel = (idx.reshape(R, 1) == nt_ids).astype(jnp.float32)
    s = (w.reshape(R, 1) * sel).sum(axis=0)
    s_bcast = jnp.broadcast_to(s.reshape(N_NT, 1), (N_NT, D))
    contrib = jnp.dot(nt_ref[...], s_bcast,
                      preferred_element_type=jnp.float32)
    g = 0.1 + 1.9 * jax.nn.sigmoid(contrib)
    o_ref[...] = x_ref[...] * g


@jax.jit
def kernel(x, nt_levels, w, idx):
    return pl.pallas_call(
        _body,
        grid=(B // BLK,),
        in_specs=[
            pl.BlockSpec((BLK, D), lambda i: (i, 0)),
            pl.BlockSpec((BLK, N_NT), lambda i: (i, 0)),
            pl.BlockSpec((1, R), lambda i: (0, 0)),
            pl.BlockSpec((1, R), lambda i: (0, 0)),
        ],
        out_specs=pl.BlockSpec((BLK, D), lambda i: (i, 0)),
        out_shape=jax.ShapeDtypeStruct((B, D), jnp.float32),
    )(x, nt_levels, w.reshape(1, R), idx.reshape(1, R))
